# baseline (speedup 1.0000x reference)
import numpy as np

import jax
import jax.numpy as jnp
from jax import lax
from jax.experimental import pallas as pl
from jax.experimental.pallas import tpu as pltpu
from jax.experimental.pallas import tpu_sc as plsc

N = 10000
E = 320000
D = 128
H = 128
OUT = 47
OUTP = 64
NP_ = N + 8
ROWS = N + 16
CW = 16

NC = 2
NS = 16
NW = NC * NS
CHUNK = 128
GRP = 8
GPW = 10
TOTG = NW * GPW
EPAD = TOTG * GRP * CHUNK
RPS = N // NS

def _perm(F):
    p = np.arange(F).reshape(F // 32, 2, 16).transpose(0, 2, 1).reshape(F)
    return np.asarray(p, np.int32)


_PERM128 = _perm(128)
_PERM64 = _perm(64)


def _make_segsum(F):
    mesh = plsc.VectorSubcoreMesh(core_axis_name="c", subcore_axis_name="s")

    def body(h_hbm, src_hbm, dst_hbm, zeros_hbm, out_hbm,
             src_v, dst_v, bufb0, bufb1, buff, acc, sem0, sem1, sem_i):
        cid = lax.axis_index("c")
        sid = lax.axis_index("s")
        r0 = sid * RPS
        bufs = (bufb0, bufb1)
        sems = (sem0, sem1)
        gb = (sid * NC + cid) * GPW

        pltpu.sync_copy(zeros_hbm, acc.at[pl.ds(r0, RPS)])
        pltpu.sync_copy(src_hbm.at[gb], src_v.at[pl.ds(0, GRP)])
        pltpu.sync_copy(dst_hbm.at[gb], dst_v.at[pl.ds(0, GRP)])
        plsc.subcore_barrier()

        pltpu.async_copy(src_hbm.at[gb + 1], src_v.at[pl.ds(GRP, GRP)], sem_i)
        pltpu.async_copy(dst_hbm.at[gb + 1], dst_v.at[pl.ds(GRP, GRP)], sem_i)
        pltpu.async_copy(h_hbm.at[src_v.at[0]], bufb0, sem0)

        def unpack_scatter(cur, drow):
            @pl.loop(0, CHUNK, unroll=4)
            def _(r):
                for blk in range(F // 32):
                    ab = bufs[cur][r, pl.ds(blk * 32, 32)]
                    a, b2 = plsc.unpack(ab, format=plsc.PackFormat.INTERLEAVED)
                    buff[r, pl.ds(blk * 32, 16)] = a
                    buff[r, pl.ds(blk * 32 + 16, 16)] = b2
            pltpu.sync_copy(buff, acc.at[drow], add=True)

        @pl.loop(0, GPW)
        def _(g):
            b = lax.rem(g, 2) * GRP
            nb = lax.rem(g + 1, 2) * GRP
            for k in range(GRP):
                cur, nxt = k % 2, (k + 1) % 2
                if k < GRP - 1:
                    pltpu.async_copy(h_hbm.at[src_v.at[b + k + 1]],
                                     bufs[nxt], sems[nxt])
                    pltpu.make_async_copy(h_hbm.at[src_v.at[b + k]],
                                          bufs[cur], sems[cur]).wait()
                    unpack_scatter(cur, dst_v.at[b + k])
                else:
                    @pl.when(g + 1 < GPW)
                    def _():
                        pltpu.make_async_copy(
                            src_hbm.at[gb + g + 1],
                            src_v.at[pl.ds(nb, GRP)], sem_i).wait()
                        pltpu.make_async_copy(
                            dst_hbm.at[gb + g + 1],
                            dst_v.at[pl.ds(nb, GRP)], sem_i).wait()
                        pltpu.async_copy(h_hbm.at[src_v.at[nb]],
                                         bufs[nxt], sems[nxt])
                    pltpu.make_async_copy(h_hbm.at[src_v.at[b + k]],
                                          bufs[cur], sems[cur]).wait()
                    unpack_scatter(cur, dst_v.at[b + k])

                    @pl.when(g + 2 < GPW)
                    def _():
                        pltpu.async_copy(src_hbm.at[gb + g + 2],
                                         src_v.at[pl.ds(b, GRP)], sem_i)
                        pltpu.async_copy(dst_hbm.at[gb + g + 2],
                                         dst_v.at[pl.ds(b, GRP)], sem_i)

        plsc.subcore_barrier()
        pltpu.sync_copy(acc.at[pl.ds(r0, RPS)],
                        out_hbm.at[cid, pl.ds(r0, RPS)])

    return pl.kernel(
        body,
        out_type=jax.ShapeDtypeStruct((NC, N, F), jnp.float32),
        mesh=mesh,
        compiler_params=pltpu.CompilerParams(use_tc_tiling_on_sc=False,
                                             needs_layout_passes=False),
        scratch_types=[
            pltpu.VMEM((2 * GRP, CHUNK), jnp.int32),
            pltpu.VMEM((2 * GRP, CHUNK), jnp.int32),
            pltpu.VMEM((CHUNK, F), jnp.bfloat16),
            pltpu.VMEM((CHUNK, F), jnp.bfloat16),
            pltpu.VMEM((CHUNK, F), jnp.float32),
            pltpu.VMEM_SHARED((ROWS, F), jnp.float32),
            pltpu.SemaphoreType.DMA,
            pltpu.SemaphoreType.DMA,
            pltpu.SemaphoreType.DMA,
        ],
    )


_segsum_h = _make_segsum(H)
_segsum_out = _make_segsum(OUTP)


def _count_body(dst_hbm, zeros_hbm, out_hbm, dst_v, ones_v, acc, sem_i):
    cid = lax.axis_index("c")
    sid = lax.axis_index("s")
    r0 = sid * RPS
    gb = (sid * NC + cid) * GPW

    pltpu.sync_copy(zeros_hbm, acc.at[pl.ds(r0, RPS)])
    pltpu.sync_copy(dst_hbm.at[gb], dst_v.at[pl.ds(0, GRP)])

    @pl.loop(0, CHUNK)
    def _(r):
        ones_v[r, pl.ds(0, CW)] = jnp.ones((CW,), jnp.float32)

    plsc.subcore_barrier()
    pltpu.async_copy(dst_hbm.at[gb + 1], dst_v.at[pl.ds(GRP, GRP)], sem_i)

    @pl.loop(0, GPW)
    def _(g):
        b = lax.rem(g, 2) * GRP
        nb = lax.rem(g + 1, 2) * GRP
        for k in range(GRP):
            pltpu.sync_copy(ones_v, acc.at[dst_v.at[b + k]], add=True)
        @pl.when(g + 1 < GPW)
        def _():
            pltpu.make_async_copy(dst_hbm.at[gb + g + 1],
                                  dst_v.at[pl.ds(nb, GRP)], sem_i).wait()
        @pl.when(g + 2 < GPW)
        def _():
            pltpu.async_copy(dst_hbm.at[gb + g + 2],
                             dst_v.at[pl.ds(b, GRP)], sem_i)

    plsc.subcore_barrier()
    pltpu.sync_copy(acc.at[pl.ds(r0, RPS)],
                    out_hbm.at[cid, pl.ds(r0, RPS)])


_count = pl.kernel(
    _count_body,
    out_type=jax.ShapeDtypeStruct((NC, N, CW), jnp.float32),
    mesh=plsc.VectorSubcoreMesh(core_axis_name="c", subcore_axis_name="s"),
    compiler_params=pltpu.CompilerParams(use_tc_tiling_on_sc=False),
    scratch_types=[
        pltpu.VMEM((2 * GRP, CHUNK), jnp.int32),
        pltpu.VMEM((CHUNK, CW), jnp.float32),
        pltpu.VMEM_SHARED((ROWS, CW), jnp.float32),
        pltpu.SemaphoreType.DMA,
    ],
)


def _mm(x, w):
    return lax.dot_general(x, w, (((1,), (1,)), ((), ())),
                           precision=lax.Precision.HIGHEST,
                           preferred_element_type=jnp.float32)


def _tc0_body(x_ref, wl_ref, h_ref):
    h_ref[:N, :] = _mm(x_ref[...], wl_ref[...]).astype(jnp.bfloat16)
    h_ref[N:, :] = jnp.zeros((NP_ - N, H), jnp.bfloat16)


_tc0 = pl.pallas_call(
    _tc0_body,
    out_shape=jax.ShapeDtypeStruct((NP_, H), jnp.bfloat16),
)


def _tc1_body(p_ref, pc_ref, x_ref, wr_ref, b_ref, g_ref, be_ref, wln_ref,
              x1_ref, h1_ref, inv_ref):
    s = p_ref[0] + p_ref[1]
    c = pc_ref[0, :, 0:1] + pc_ref[1, :, 0:1]
    inv = 1.0 / jnp.maximum(c, 1.0)
    z = s * inv + _mm(x_ref[...], wr_ref[...]) + b_ref[...][None, :]
    mu = jnp.mean(z, axis=0, keepdims=True)
    var = jnp.mean((z - mu) ** 2, axis=0, keepdims=True)
    zn = (z - mu) / jnp.sqrt(var + 1e-5) * g_ref[...][None, :] + be_ref[...][None, :]
    x1 = jnp.maximum(zn, 0.0)
    x1_ref[...] = x1
    h1_ref[:N, :] = _mm(x1, wln_ref[...]).astype(jnp.bfloat16)
    h1_ref[N:, :] = jnp.zeros((NP_ - N, H), jnp.bfloat16)
    inv_ref[...] = inv


_tc1 = pl.pallas_call(
    _tc1_body,
    out_shape=(
        jax.ShapeDtypeStruct((N, H), jnp.float32),
        jax.ShapeDtypeStruct((NP_, H), jnp.bfloat16),
        jax.ShapeDtypeStruct((N, 1), jnp.float32),
    ),
)


def _tc2_body(p_ref, x_ref, inv_ref, wr_ref, b_ref, g_ref, be_ref, wln_ref,
              x2_ref, h2_ref):
    s = p_ref[0] + p_ref[1]
    z = s * inv_ref[...] + _mm(x_ref[...], wr_ref[...]) + b_ref[...][None, :]
    mu = jnp.mean(z, axis=0, keepdims=True)
    var = jnp.mean((z - mu) ** 2, axis=0, keepdims=True)
    zn = (z - mu) / jnp.sqrt(var + 1e-5) * g_ref[...][None, :] + be_ref[...][None, :]
    x2 = jnp.maximum(zn, 0.0)
    x2_ref[...] = x2
    h2_ref[:N, :] = _mm(x2, wln_ref[...]).astype(jnp.bfloat16)
    h2_ref[N:, :] = jnp.zeros((NP_ - N, OUTP), jnp.bfloat16)


_tc2 = pl.pallas_call(
    _tc2_body,
    out_shape=(
        jax.ShapeDtypeStruct((N, H), jnp.float32),
        jax.ShapeDtypeStruct((NP_, OUTP), jnp.bfloat16),
    ),
)


def _tc3_body(p_ref, x_ref, inv_ref, wr_ref, b_ref, out_ref):
    s = p_ref[0, :, :OUT] + p_ref[1, :, :OUT]
    out_ref[...] = s * inv_ref[...] + _mm(x_ref[...], wr_ref[...]) + b_ref[...][None, :]


_tc3 = pl.pallas_call(
    _tc3_body,
    out_shape=jax.ShapeDtypeStruct((N, OUT), jnp.float32),
)


def kernel(features, edge_index, Wl0, Wr0, b0, g0, be0,
           Wl1, Wr1, b1, g1, be1, Wl2, Wr2, b2):
    src = edge_index[0]
    dst = edge_index[1]
    pad = EPAD - E
    src_p = jnp.concatenate(
        [src, jnp.full((pad,), N, jnp.int32)]).reshape(TOTG, GRP, CHUNK)
    dst_p = jnp.concatenate(
        [dst, jnp.full((pad,), N, jnp.int32)]).reshape(TOTG, GRP, CHUNK)
    wl0p = Wl0[_PERM128, :]
    wl1p = Wl1[_PERM128, :]
    wl2p = jnp.concatenate(
        [Wl2, jnp.zeros((OUTP - OUT, H), jnp.float32)], axis=0)[_PERM64, :]

    zh = jnp.zeros((RPS, H), jnp.float32)
    zo = jnp.zeros((RPS, OUTP), jnp.float32)
    zc = jnp.zeros((RPS, CW), jnp.float32)

    pc = _count(dst_p, zc)
    h0 = _tc0(features, wl0p)
    p0 = _segsum_h(h0, src_p, dst_p, zh)
    x1, h1, inv = _tc1(p0, pc, features, Wr0, b0, g0, be0, wl1p)
    p1 = _segsum_h(h1, src_p, dst_p, zh)
    x2, h2 = _tc2(p1, x1, inv, Wr1, b1, g1, be1, wl2p)
    p2 = _segsum_out(h2, src_p, dst_p, zo)
    out = _tc3(p2, x2, inv, Wr2, b2)
    return out


# speedup vs baseline: 1.0047x; 1.0047x over previous
import numpy as np

import jax
import jax.numpy as jnp
from jax import lax
from jax.experimental import pallas as pl
from jax.experimental.pallas import tpu as pltpu
from jax.experimental.pallas import tpu_sc as plsc

N = 10000
E = 320000
D = 128
H = 128
OUT = 47
OUTP = 64
NP_ = N + 8
ROWS = N + 16
CW = 16

NC = 2
NS = 16
NW = NC * NS
CHUNK = 128
GRP = 8
GPW = 10
TOTG = NW * GPW
EPAD = TOTG * GRP * CHUNK
RPS = N // NS

def _perm(F):
    p = np.arange(F).reshape(F // 32, 2, 16).transpose(0, 2, 1).reshape(F)
    return np.asarray(p, np.int32)


_PERM128 = _perm(128)
_PERM64 = _perm(64)


def _make_segsum(F):
    mesh = plsc.VectorSubcoreMesh(core_axis_name="c", subcore_axis_name="s")

    def body(h_hbm, src_hbm, dst_hbm, zeros_hbm, out_hbm,
             src_v, dst_v, bufb0, bufb1, buff, acc, sem0, sem1, sem_i):
        cid = lax.axis_index("c")
        sid = lax.axis_index("s")
        r0 = sid * RPS
        bufs = (bufb0, bufb1)
        sems = (sem0, sem1)
        gb = (sid * NC + cid) * GPW

        pltpu.sync_copy(zeros_hbm, acc.at[pl.ds(r0, RPS)])
        pltpu.sync_copy(src_hbm.at[gb], src_v.at[pl.ds(0, GRP)])
        pltpu.sync_copy(dst_hbm.at[gb], dst_v.at[pl.ds(0, GRP)])
        plsc.subcore_barrier()

        pltpu.async_copy(src_hbm.at[gb + 1], src_v.at[pl.ds(GRP, GRP)], sem_i)
        pltpu.async_copy(dst_hbm.at[gb + 1], dst_v.at[pl.ds(GRP, GRP)], sem_i)
        pltpu.async_copy(h_hbm.at[src_v.at[0]], bufb0, sem0)

        def unpack_scatter(cur, drow):
            @pl.loop(0, CHUNK)
            def _(r):
                for blk in range(F // 32):
                    ab = bufs[cur][r, pl.ds(blk * 32, 32)]
                    a, b2 = plsc.unpack(ab, format=plsc.PackFormat.INTERLEAVED)
                    buff[r, pl.ds(blk * 32, 16)] = a
                    buff[r, pl.ds(blk * 32 + 16, 16)] = b2
            pltpu.sync_copy(buff, acc.at[drow], add=True)

        @pl.loop(0, GPW)
        def _(g):
            b = lax.rem(g, 2) * GRP
            nb = lax.rem(g + 1, 2) * GRP
            for k in range(GRP):
                cur, nxt = k % 2, (k + 1) % 2
                if k < GRP - 1:
                    pltpu.async_copy(h_hbm.at[src_v.at[b + k + 1]],
                                     bufs[nxt], sems[nxt])
                    pltpu.make_async_copy(h_hbm.at[src_v.at[b + k]],
                                          bufs[cur], sems[cur]).wait()
                    unpack_scatter(cur, dst_v.at[b + k])
                else:
                    @pl.when(g + 1 < GPW)
                    def _():
                        pltpu.make_async_copy(
                            src_hbm.at[gb + g + 1],
                            src_v.at[pl.ds(nb, GRP)], sem_i).wait()
                        pltpu.make_async_copy(
                            dst_hbm.at[gb + g + 1],
                            dst_v.at[pl.ds(nb, GRP)], sem_i).wait()
                        pltpu.async_copy(h_hbm.at[src_v.at[nb]],
                                         bufs[nxt], sems[nxt])
                    pltpu.make_async_copy(h_hbm.at[src_v.at[b + k]],
                                          bufs[cur], sems[cur]).wait()
                    unpack_scatter(cur, dst_v.at[b + k])

                    @pl.when(g + 2 < GPW)
                    def _():
                        pltpu.async_copy(src_hbm.at[gb + g + 2],
                                         src_v.at[pl.ds(b, GRP)], sem_i)
                        pltpu.async_copy(dst_hbm.at[gb + g + 2],
                                         dst_v.at[pl.ds(b, GRP)], sem_i)

        plsc.subcore_barrier()
        pltpu.sync_copy(acc.at[pl.ds(r0, RPS)],
                        out_hbm.at[cid, pl.ds(r0, RPS)])

    return pl.kernel(
        body,
        out_type=jax.ShapeDtypeStruct((NC, N, F), jnp.float32),
        mesh=mesh,
        compiler_params=pltpu.CompilerParams(use_tc_tiling_on_sc=False,
                                             needs_layout_passes=False),
        scratch_types=[
            pltpu.VMEM((2 * GRP, CHUNK), jnp.int32),
            pltpu.VMEM((2 * GRP, CHUNK), jnp.int32),
            pltpu.VMEM((CHUNK, F), jnp.bfloat16),
            pltpu.VMEM((CHUNK, F), jnp.bfloat16),
            pltpu.VMEM((CHUNK, F), jnp.float32),
            pltpu.VMEM_SHARED((ROWS, F), jnp.float32),
            pltpu.SemaphoreType.DMA,
            pltpu.SemaphoreType.DMA,
            pltpu.SemaphoreType.DMA,
        ],
    )


_segsum_h = _make_segsum(H)
_segsum_out = _make_segsum(OUTP)


def _count_body(dst_hbm, zeros_hbm, out_hbm, dst_v, ones_v, acc, sem_i):
    cid = lax.axis_index("c")
    sid = lax.axis_index("s")
    r0 = sid * RPS
    gb = (sid * NC + cid) * GPW

    pltpu.sync_copy(zeros_hbm, acc.at[pl.ds(r0, RPS)])
    pltpu.sync_copy(dst_hbm.at[gb], dst_v.at[pl.ds(0, GRP)])

    @pl.loop(0, CHUNK)
    def _(r):
        ones_v[r, pl.ds(0, CW)] = jnp.ones((CW,), jnp.float32)

    plsc.subcore_barrier()
    pltpu.async_copy(dst_hbm.at[gb + 1], dst_v.at[pl.ds(GRP, GRP)], sem_i)

    @pl.loop(0, GPW)
    def _(g):
        b = lax.rem(g, 2) * GRP
        nb = lax.rem(g + 1, 2) * GRP
        for k in range(GRP):
            pltpu.sync_copy(ones_v, acc.at[dst_v.at[b + k]], add=True)
        @pl.when(g + 1 < GPW)
        def _():
            pltpu.make_async_copy(dst_hbm.at[gb + g + 1],
                                  dst_v.at[pl.ds(nb, GRP)], sem_i).wait()
        @pl.when(g + 2 < GPW)
        def _():
            pltpu.async_copy(dst_hbm.at[gb + g + 2],
                             dst_v.at[pl.ds(b, GRP)], sem_i)

    plsc.subcore_barrier()
    pltpu.sync_copy(acc.at[pl.ds(r0, RPS)],
                    out_hbm.at[cid, pl.ds(r0, RPS)])


_count = pl.kernel(
    _count_body,
    out_type=jax.ShapeDtypeStruct((NC, N, CW), jnp.float32),
    mesh=plsc.VectorSubcoreMesh(core_axis_name="c", subcore_axis_name="s"),
    compiler_params=pltpu.CompilerParams(use_tc_tiling_on_sc=False),
    scratch_types=[
        pltpu.VMEM((2 * GRP, CHUNK), jnp.int32),
        pltpu.VMEM((CHUNK, CW), jnp.float32),
        pltpu.VMEM_SHARED((ROWS, CW), jnp.float32),
        pltpu.SemaphoreType.DMA,
    ],
)


def _mm(x, w):
    return lax.dot_general(x, w, (((1,), (1,)), ((), ())),
                           precision=lax.Precision.HIGHEST,
                           preferred_element_type=jnp.float32)


def _tc0_body(x_ref, wl_ref, h_ref):
    h_ref[:N, :] = _mm(x_ref[...], wl_ref[...]).astype(jnp.bfloat16)
    h_ref[N:, :] = jnp.zeros((NP_ - N, H), jnp.bfloat16)


_tc0 = pl.pallas_call(
    _tc0_body,
    out_shape=jax.ShapeDtypeStruct((NP_, H), jnp.bfloat16),
)


def _tc1_body(p_ref, pc_ref, x_ref, wr_ref, b_ref, g_ref, be_ref, wln_ref,
              x1_ref, h1_ref, inv_ref):
    s = p_ref[0] + p_ref[1]
    c = pc_ref[0, :, 0:1] + pc_ref[1, :, 0:1]
    inv = 1.0 / jnp.maximum(c, 1.0)
    z = s * inv + _mm(x_ref[...], wr_ref[...]) + b_ref[...][None, :]
    mu = jnp.mean(z, axis=0, keepdims=True)
    var = jnp.mean((z - mu) ** 2, axis=0, keepdims=True)
    zn = (z - mu) / jnp.sqrt(var + 1e-5) * g_ref[...][None, :] + be_ref[...][None, :]
    x1 = jnp.maximum(zn, 0.0)
    x1_ref[...] = x1
    h1_ref[:N, :] = _mm(x1, wln_ref[...]).astype(jnp.bfloat16)
    h1_ref[N:, :] = jnp.zeros((NP_ - N, H), jnp.bfloat16)
    inv_ref[...] = inv


_tc1 = pl.pallas_call(
    _tc1_body,
    out_shape=(
        jax.ShapeDtypeStruct((N, H), jnp.float32),
        jax.ShapeDtypeStruct((NP_, H), jnp.bfloat16),
        jax.ShapeDtypeStruct((N, 1), jnp.float32),
    ),
)


def _tc2_body(p_ref, x_ref, inv_ref, wr_ref, b_ref, g_ref, be_ref, wln_ref,
              x2_ref, h2_ref):
    s = p_ref[0] + p_ref[1]
    z = s * inv_ref[...] + _mm(x_ref[...], wr_ref[...]) + b_ref[...][None, :]
    mu = jnp.mean(z, axis=0, keepdims=True)
    var = jnp.mean((z - mu) ** 2, axis=0, keepdims=True)
    zn = (z - mu) / jnp.sqrt(var + 1e-5) * g_ref[...][None, :] + be_ref[...][None, :]
    x2 = jnp.maximum(zn, 0.0)
    x2_ref[...] = x2
    h2_ref[:N, :] = _mm(x2, wln_ref[...]).astype(jnp.bfloat16)
    h2_ref[N:, :] = jnp.zeros((NP_ - N, OUTP), jnp.bfloat16)


_tc2 = pl.pallas_call(
    _tc2_body,
    out_shape=(
        jax.ShapeDtypeStruct((N, H), jnp.float32),
        jax.ShapeDtypeStruct((NP_, OUTP), jnp.bfloat16),
    ),
)


def _tc3_body(p_ref, x_ref, inv_ref, wr_ref, b_ref, out_ref):
    s = p_ref[0, :, :OUT] + p_ref[1, :, :OUT]
    out_ref[...] = s * inv_ref[...] + _mm(x_ref[...], wr_ref[...]) + b_ref[...][None, :]


_tc3 = pl.pallas_call(
    _tc3_body,
    out_shape=jax.ShapeDtypeStruct((N, OUT), jnp.float32),
)


def kernel(features, edge_index, Wl0, Wr0, b0, g0, be0,
           Wl1, Wr1, b1, g1, be1, Wl2, Wr2, b2):
    src = edge_index[0]
    dst = edge_index[1]
    pad = EPAD - E
    src_p = jnp.concatenate(
        [src, jnp.full((pad,), N, jnp.int32)]).reshape(TOTG, GRP, CHUNK)
    dst_p = jnp.concatenate(
        [dst, jnp.full((pad,), N, jnp.int32)]).reshape(TOTG, GRP, CHUNK)
    wl0p = Wl0[_PERM128, :]
    wl1p = Wl1[_PERM128, :]
    wl2p = jnp.concatenate(
        [Wl2, jnp.zeros((OUTP - OUT, H), jnp.float32)], axis=0)[_PERM64, :]

    zh = jnp.zeros((RPS, H), jnp.float32)
    zo = jnp.zeros((RPS, OUTP), jnp.float32)
    zc = jnp.zeros((RPS, CW), jnp.float32)

    pc = _count(dst_p, zc)
    h0 = _tc0(features, wl0p)
    p0 = _segsum_h(h0, src_p, dst_p, zh)
    x1, h1, inv = _tc1(p0, pc, features, Wr0, b0, g0, be0, wl1p)
    p1 = _segsum_h(h1, src_p, dst_p, zh)
    x2, h2 = _tc2(p1, x1, inv, Wr1, b1, g1, be1, wl2p)
    p2 = _segsum_out(h2, src_p, dst_p, zo)
    out = _tc3(p2, x2, inv, Wr2, b2)
    return out
